# baseline (speedup 1.0000x reference)
import functools

import jax
import jax.numpy as jnp
from jax import lax
from jax.experimental import pallas as pl
from jax.experimental.pallas import tpu as pltpu
from jax.experimental.pallas import tpu_sc as plsc

_VOCAB = 1000000
_D = 100
_DP = 104
_B = 16384
_S = 40

_NC, _NS = 2, 16
_NW = _NC * _NS
_BPW = _B // _NW
_G = 8
_NGROUPS = _BPW // _G
_RPG = _G * _S
_CHUNK = 80
_NCH = _RPG // _CHUNK
_XROWS_PER_W = _BPW * _S // _CHUNK

_LANES = 16
_OFFS = (0, 16, 32, 48, 64, 80, 84)


def _gather_group(x_hbm, table_hbm, idx_ref, rows_ref, sem, xrow):
    pltpu.sync_copy(x_hbm.at[pl.ds(xrow, _NCH)], idx_ref)
    for c in range(_NCH):
        pltpu.async_copy(
            table_hbm.at[idx_ref.at[c]],
            rows_ref.at[pl.ds(c * _CHUNK, _CHUNK)],
            sem,
        )


def _wait_group(table_hbm, idx_ref, rows_ref, sem):
    for c in range(_NCH):
        pltpu.make_async_copy(
            table_hbm.at[idx_ref.at[c]],
            rows_ref.at[pl.ds(c * _CHUNK, _CHUNK)],
            sem,
        ).wait()


def _compute_group(rows_ref, out_ref):
    inv = jnp.full((_LANES,), 1.0 / _S, dtype=jnp.float32)

    def batch_body(b, carry):
        base = b * _S

        def s_body(s, accs):
            r = base + s
            return tuple(
                accs[j] + rows_ref[r, pl.ds(_OFFS[j], _LANES)]
                for j in range(len(_OFFS))
            )

        zeros = tuple(
            jnp.zeros((_LANES,), jnp.float32) for _ in range(len(_OFFS))
        )
        accs = lax.fori_loop(0, _S, s_body, zeros)
        for j in range(len(_OFFS)):
            out_ref[pl.ds(b * _D + _OFFS[j], _LANES)] = accs[j] * inv
        return carry

    lax.fori_loop(0, _G, batch_body, 0)


def _swem_body(x_hbm, table_hbm, out_hbm, idx_v, rows_v, out_v, sem0, sem1):
    wid = lax.axis_index("s") * _NC + lax.axis_index("c")
    xrow0 = wid * _XROWS_PER_W
    b0 = wid * _BPW
    sems = (sem0, sem1)

    for slot in range(2):
        _gather_group(
            x_hbm, table_hbm, idx_v.at[slot], rows_v.at[slot], sems[slot],
            xrow0 + slot * _NCH,
        )

    def loop_body(k, carry):
        g2 = k * 2
        for slot in range(2):
            g = g2 + slot
            _wait_group(table_hbm, idx_v.at[slot], rows_v.at[slot], sems[slot])
            _compute_group(rows_v.at[slot], out_v)
            pltpu.sync_copy(
                out_v, out_hbm.at[pl.ds((b0 + g * _G) * _D, _G * _D)])

            @pl.when(g + 2 < _NGROUPS)
            def _():
                _gather_group(
                    x_hbm, table_hbm, idx_v.at[slot], rows_v.at[slot],
                    sems[slot], xrow0 + (g + 2) * _NCH,
                )

        return carry

    lax.fori_loop(0, _NGROUPS // 2, loop_body, 0)


@jax.jit
def kernel(x, lengths, table):
    del lengths
    x2 = x.astype(jnp.int32).reshape(_B * _S // _CHUNK, _CHUNK)
    tpad = jnp.pad(table, ((0, 0), (0, _DP - _D)))

    mesh = plsc.VectorSubcoreMesh(core_axis_name="c", subcore_axis_name="s")
    fn = pl.kernel(
        _swem_body,
        out_type=jax.ShapeDtypeStruct((_B * _D,), jnp.float32),
        mesh=mesh,
        scratch_types=[
            pltpu.VMEM((2, _NCH, _CHUNK), jnp.int32),
            pltpu.VMEM((2, _RPG, _DP), jnp.float32),
            pltpu.VMEM((_G * _D,), jnp.float32),
            pltpu.SemaphoreType.DMA,
            pltpu.SemaphoreType.DMA,
        ],
        compiler_params=pltpu.CompilerParams(use_tc_tiling_on_sc=False),
    )
    return fn(x2, tpad).reshape(_B, _D)


# speedup vs baseline: 1.0008x; 1.0008x over previous
import functools

import jax
import jax.numpy as jnp
from jax import lax
from jax.experimental import pallas as pl
from jax.experimental.pallas import tpu as pltpu
from jax.experimental.pallas import tpu_sc as plsc

_VOCAB = 1000000
_D = 100
_DP = 104
_B = 16384
_S = 40

_NC, _NS = 2, 16
_NW = _NC * _NS
_BPW = _B // _NW
_G = 8
_NGROUPS = _BPW // _G
_RPG = _G * _S
_CHUNK = 80
_NCH = _RPG // _CHUNK
_XROWS_PER_W = _BPW * _S // _CHUNK

_LANES = 16
_OFFS = (0, 16, 32, 48, 64, 80, 84)


def _gather_group(x_hbm, table_hbm, idx_ref, rows_ref, sem, xrow):
    pltpu.sync_copy(x_hbm.at[pl.ds(xrow, _NCH)], idx_ref)
    for c in range(_NCH):
        pltpu.async_copy(
            table_hbm.at[idx_ref.at[c]],
            rows_ref.at[pl.ds(c * _CHUNK, _CHUNK)],
            sem,
        )


def _wait_group(table_hbm, idx_ref, rows_ref, sem):
    for c in range(_NCH):
        pltpu.make_async_copy(
            table_hbm.at[idx_ref.at[c]],
            rows_ref.at[pl.ds(c * _CHUNK, _CHUNK)],
            sem,
        ).wait()


def _compute_group(rows_ref, out_ref):
    inv = jnp.full((_LANES,), 1.0 / _S, dtype=jnp.float32)

    def batch_body(b, carry):
        base = b * _S

        def s_body(s, accs):
            r = base + s
            return tuple(
                accs[j] + rows_ref[r, pl.ds(_OFFS[j], _LANES)]
                for j in range(len(_OFFS))
            )

        zeros = tuple(
            jnp.zeros((_LANES,), jnp.float32) for _ in range(len(_OFFS))
        )
        accs = lax.fori_loop(0, _S, s_body, zeros)
        for j in range(len(_OFFS)):
            out_ref[pl.ds(b * _D + _OFFS[j], _LANES)] = accs[j] * inv
        return carry

    lax.fori_loop(0, _G, batch_body, 0)


def _swem_body(x_hbm, table_hbm, out_hbm, idx_v, rows_v, out_v, sem0, sem1):
    wid = lax.axis_index("s") * _NC + lax.axis_index("c")
    xrow0 = wid * _XROWS_PER_W
    b0 = wid * _BPW
    sems = (sem0, sem1)

    for slot in range(2):
        _gather_group(
            x_hbm, table_hbm, idx_v.at[slot], rows_v.at[slot], sems[slot],
            xrow0 + slot * _NCH,
        )

    def loop_body(k, carry):
        g2 = k * 2
        for slot in range(2):
            g = g2 + slot
            _wait_group(table_hbm, idx_v.at[slot], rows_v.at[slot], sems[slot])
            _compute_group(rows_v.at[slot], out_v)
            pltpu.sync_copy(
                out_v, out_hbm.at[pl.ds((b0 + g * _G) * _D, _G * _D)])

            @pl.when(g + 2 < _NGROUPS)
            def _():
                _gather_group(
                    x_hbm, table_hbm, idx_v.at[slot], rows_v.at[slot],
                    sems[slot], xrow0 + (g + 2) * _NCH,
                )

        return carry

    lax.fori_loop(0, _NGROUPS // 2, loop_body, 0)


@jax.jit
def kernel(x, lengths, table):
    del lengths
    x2 = x.astype(jnp.int32).reshape(_B * _S // _CHUNK, _CHUNK)
    tpad = jnp.concatenate(
        [table, jnp.zeros((_VOCAB, _DP - _D), jnp.float32)], axis=1)

    mesh = plsc.VectorSubcoreMesh(core_axis_name="c", subcore_axis_name="s")
    fn = pl.kernel(
        _swem_body,
        out_type=jax.ShapeDtypeStruct((_B * _D,), jnp.float32),
        mesh=mesh,
        scratch_types=[
            pltpu.VMEM((2, _NCH, _CHUNK), jnp.int32),
            pltpu.VMEM((2, _RPG, _DP), jnp.float32),
            pltpu.VMEM((_G * _D,), jnp.float32),
            pltpu.SemaphoreType.DMA,
            pltpu.SemaphoreType.DMA,
        ],
        compiler_params=pltpu.CompilerParams(use_tc_tiling_on_sc=False),
    )
    return fn(x2, tpad).reshape(_B, _D)
